# Initial kernel scaffold; baseline (speedup 1.0000x reference)
import jax
import jax.numpy as jnp
from jax.experimental import pallas as pl


def kernel(x, edge_index, batch, W1, b1, W2, b2, W3, b3, lin_W, lin_b):
    raise NotImplementedError("write your pallas kernel here")


# speedup vs baseline: 16.6281x; 16.6281x over previous
import functools

import jax
import jax.numpy as jnp
from jax import lax
from jax.experimental import pallas as pl
from jax.experimental.pallas import tpu as pltpu
from jax.experimental.pallas import tpu_sc as plsc

N = 10000
E = 320000
D = 128
G = 16
NC, NS = 2, 16
NW = NC * NS
EPW = E // NW
CHUNK = 80
NCHUNK = EPW // CHUNK
ROWS_A = 632
ROWS_LAST = N - (NS - 1) * ROWS_A
DEG_PS = 640
DEGN = NS * DEG_PS
DEG_F = 16

_MESH = plsc.VectorSubcoreMesh(
    core_axis_name="c", subcore_axis_name="s", num_cores=NC, num_subcores=NS)


@functools.partial(
    pl.kernel,
    out_type=jax.ShapeDtypeStruct((NC, DEGN, DEG_F), jnp.float32),
    mesh=_MESH,
    scratch_types=[
        pltpu.VMEM_SHARED((DEGN, DEG_F), jnp.float32),
        pltpu.VMEM((NCHUNK, CHUNK), jnp.int32),
        pltpu.VMEM((CHUNK, DEG_F), jnp.float32),
    ],
)
def _deg_kernel(dst_hbm, ones_hbm, zeros_hbm, out_hbm, acc, didx, ones_v):
    c = lax.axis_index("c")
    s = lax.axis_index("s")
    wid = s * NC + c
    pltpu.sync_copy(zeros_hbm, acc.at[pl.ds(s * DEG_PS, DEG_PS)])
    pltpu.sync_copy(ones_hbm, ones_v)
    pltpu.sync_copy(dst_hbm.at[wid], didx)
    plsc.subcore_barrier()

    def body(j, carry):
        pltpu.sync_copy(ones_v, acc.at[didx.at[j]], add=True)
        return carry

    lax.fori_loop(0, NCHUNK, body, 0)
    plsc.subcore_barrier()
    pltpu.sync_copy(acc.at[pl.ds(s * DEG_PS, DEG_PS)],
                    out_hbm.at[c, pl.ds(s * DEG_PS, DEG_PS)])


@functools.partial(
    pl.kernel,
    out_type=jax.ShapeDtypeStruct((NC, N, D), jnp.float32),
    mesh=_MESH,
    scratch_types=[
        pltpu.VMEM_SHARED((N, D), jnp.float32),
        pltpu.VMEM((NCHUNK, CHUNK), jnp.int32),
        pltpu.VMEM((NCHUNK, CHUNK), jnp.int32),
        pltpu.VMEM((CHUNK, D), jnp.float32),
        pltpu.SemaphoreType.DMA,
    ],
)
def _scatter_kernel(t_hbm, src_hbm, dst_hbm, out_hbm, acc, sidx, didx, rows, sem):
    c = lax.axis_index("c")
    s = lax.axis_index("s")
    wid = s * NC + c
    base = s * ROWS_A

    @pl.when(s < NS - 1)
    def _():
        pltpu.sync_copy(t_hbm.at[pl.ds(base, ROWS_A)],
                        acc.at[pl.ds(base, ROWS_A)])

    @pl.when(s == NS - 1)
    def _():
        pltpu.sync_copy(t_hbm.at[pl.ds(base, ROWS_LAST)],
                        acc.at[pl.ds(base, ROWS_LAST)])

    pltpu.sync_copy(src_hbm.at[wid], sidx)
    pltpu.sync_copy(dst_hbm.at[wid], didx)
    plsc.subcore_barrier()

    def body(j, carry):
        pltpu.async_copy(t_hbm.at[sidx.at[j]], rows, sem).wait()
        pltpu.sync_copy(rows, acc.at[didx.at[j]], add=True)
        return carry

    lax.fori_loop(0, NCHUNK, body, 0)
    plsc.subcore_barrier()

    @pl.when(s < NS - 1)
    def _():
        pltpu.sync_copy(acc.at[pl.ds(base, ROWS_A)],
                        out_hbm.at[c, pl.ds(base, ROWS_A)])

    @pl.when(s == NS - 1)
    def _():
        pltpu.sync_copy(acc.at[pl.ds(base, ROWS_LAST)],
                        out_hbm.at[c, pl.ds(base, ROWS_LAST)])


def _mm1_body(x_ref, w_ref, o_ref):
    o_ref[...] = jnp.dot(x_ref[...], w_ref[...],
                         preferred_element_type=jnp.float32)


def _prep_body(degp_ref, m_ref, t_ref, dinv_ref):
    deg2d = degp_ref[0] + degp_ref[1]
    deg = jnp.sum(deg2d, axis=1, keepdims=True) + 1.0
    dinv = lax.rsqrt(deg)[:N]
    dinv_ref[...] = dinv
    t_ref[...] = m_ref[...] * dinv


def _mid_body(p_ref, tprev_ref, dinv_ref, b_ref, w_ref, t_ref):
    dinv = dinv_ref[...]
    h = jax.nn.relu((p_ref[0] + p_ref[1] - tprev_ref[...]) * dinv + b_ref[...])
    t_ref[...] = jnp.dot(h, w_ref[...], preferred_element_type=jnp.float32) * dinv


def _final_body(p_ref, tprev_ref, dinv_ref, b_ref, batch_ref, lw_ref, lb_ref,
                o_ref):
    h = jax.nn.relu((p_ref[0] + p_ref[1] - tprev_ref[...]) * dinv_ref[...]
                    + b_ref[...])
    gids = lax.broadcasted_iota(jnp.int32, (N, G), 1)
    onehot = (batch_ref[...] == gids).astype(jnp.float32)
    summed = lax.dot_general(onehot, h, (((0,), (0,)), ((), ())),
                             preferred_element_type=jnp.float32)
    counts = jnp.sum(onehot, axis=0, keepdims=True)
    pooled = summed / jnp.maximum(counts, 1.0).T
    o_ref[...] = jnp.dot(pooled, lw_ref[...],
                         preferred_element_type=jnp.float32) + lb_ref[...]


def kernel(x, edge_index, batch, W1, b1, W2, b2, W3, b3, lin_W, lin_b):
    src = edge_index[0].reshape(NW, NCHUNK, CHUNK)
    dst = edge_index[1].reshape(NW, NCHUNK, CHUNK)
    ones_c = jnp.ones((CHUNK, DEG_F), jnp.float32)
    zeros_c = jnp.zeros((DEG_PS, DEG_F), jnp.float32)

    degp = _deg_kernel(dst, ones_c, zeros_c)

    m1 = pl.pallas_call(
        _mm1_body,
        out_shape=jax.ShapeDtypeStruct((N, D), jnp.float32),
    )(x, W1)

    t1, dinv = pl.pallas_call(
        _prep_body,
        out_shape=[jax.ShapeDtypeStruct((N, D), jnp.float32),
                   jax.ShapeDtypeStruct((N, 1), jnp.float32)],
    )(degp, m1)

    p1 = _scatter_kernel(t1, src, dst)
    t2 = pl.pallas_call(
        _mid_body,
        out_shape=jax.ShapeDtypeStruct((N, D), jnp.float32),
    )(p1, t1, dinv, b1.reshape(1, D), W2)

    p2 = _scatter_kernel(t2, src, dst)
    t3 = pl.pallas_call(
        _mid_body,
        out_shape=jax.ShapeDtypeStruct((N, D), jnp.float32),
    )(p2, t2, dinv, b2.reshape(1, D), W3)

    p3 = _scatter_kernel(t3, src, dst)
    out = pl.pallas_call(
        _final_body,
        out_shape=jax.ShapeDtypeStruct((G, 2), jnp.float32),
    )(p3, t3, dinv, b3.reshape(1, D), batch.reshape(N, 1), lin_W,
      lin_b.reshape(1, 2))
    return out
